# baseline (speedup 1.0000x reference)
import jax
import jax.numpy as jnp
from jax import lax
from jax.experimental import pallas as pl

_DN = ('NCHW', 'OIHW', 'NCHW')


def _conv(x, w, b, stride):
    y = lax.conv_general_dilated(x, w, (stride, stride), [(1, 1), (1, 1)],
                                 dimension_numbers=_DN)
    return y + b[None, :, None, None]


def _convT(x, w, b):
    wt = jnp.transpose(jnp.flip(w, axis=(2, 3)), (1, 0, 2, 3))
    y = lax.conv_general_dilated(x, wt, (1, 1), [(2, 2), (2, 2)],
                                 lhs_dilation=(2, 2), dimension_numbers=_DN)
    return y + b[None, :, None, None]


def _vq_body(zf_ref, z_sq_ref, emb_ref, emb_sq_ref, q_ref):
    zf = zf_ref[...]
    emb = emb_ref[...]
    z_sq = z_sq_ref[...]
    emb_sq = emb_sq_ref[...]
    zdote = lax.dot_general(zf, emb, (((1,), (1,)), ((), ())),
                            preferred_element_type=jnp.float32)
    d = (z_sq - 2.0 * zdote) + emb_sq
    dmin = jnp.min(d, axis=1, keepdims=True)
    ids = lax.broadcasted_iota(jnp.int32, d.shape, 1)
    big = jnp.int32(d.shape[1] + 1)
    idx = jnp.min(jnp.where(d == dmin, ids, big), axis=1, keepdims=True)
    onehot = (ids == idx).astype(jnp.float32)
    q_ref[...] = jnp.dot(onehot, emb, preferred_element_type=jnp.float32)


def _quantize(z_flat, emb):
    r, dim = z_flat.shape
    k = emb.shape[0]
    blk = 224
    z_sq = (z_flat ** 2).sum(1, keepdims=True)
    emb_sq = (emb ** 2).sum(1)[None, :]
    return pl.pallas_call(
        _vq_body,
        grid=(r // blk,),
        in_specs=[
            pl.BlockSpec((blk, dim), lambda i: (i, 0)),
            pl.BlockSpec((blk, 1), lambda i: (i, 0)),
            pl.BlockSpec((k, dim), lambda i: (0, 0)),
            pl.BlockSpec((1, k), lambda i: (0, 0)),
        ],
        out_specs=pl.BlockSpec((blk, dim), lambda i: (i, 0)),
        out_shape=jax.ShapeDtypeStruct((r, dim), jnp.float32),
    )(z_flat, z_sq, emb, emb_sq)


def kernel(x, w1, b1, w2, b2, w3, b3, w4, b4, emb,
           dw1, db1, dw2, db2, dw3, db3, dw4, db4):
    z = jax.nn.relu(_conv(x, w1, b1, 2))
    z = jax.nn.relu(_conv(z, w2, b2, 2))
    z = jax.nn.relu(_conv(z, w3, b3, 2))
    z = jax.nn.relu(_conv(z, w4, b4, 2))
    zp = jnp.transpose(z, (0, 2, 3, 1))
    z_flat = zp.reshape(-1, emb.shape[1])
    quantized = _quantize(z_flat, emb)
    quantized = jnp.transpose(quantized.reshape(zp.shape), (0, 3, 1, 2))
    y = jax.nn.relu(_convT(quantized, dw1, db1))
    y = jax.nn.relu(_convT(y, dw2, db2))
    y = jax.nn.relu(_convT(y, dw3, db3))
    y = jnp.tanh(_convT(y, dw4, db4))
    return y


# speedup vs baseline: 1.4717x; 1.4717x over previous
import functools

import jax
import jax.numpy as jnp
from jax import lax
from jax.experimental import pallas as pl

_DN = ('NCHW', 'OIHW', 'NCHW')


def _conv(x, w, b, stride):
    y = lax.conv_general_dilated(x, w, (stride, stride), [(1, 1), (1, 1)],
                                 dimension_numbers=_DN)
    return y + b[None, :, None, None]


def _vq_body(zf_ref, z_sq_ref, emb_ref, emb_sq_ref, q_ref):
    zf = zf_ref[...]
    emb = emb_ref[...]
    z_sq = z_sq_ref[...]
    emb_sq = emb_sq_ref[...]
    zdote = lax.dot_general(zf, emb, (((1,), (1,)), ((), ())),
                            preferred_element_type=jnp.float32)
    d = (z_sq - 2.0 * zdote) + emb_sq
    dmin = jnp.min(d, axis=1, keepdims=True)
    ids = lax.broadcasted_iota(jnp.int32, d.shape, 1)
    big = jnp.int32(d.shape[1] + 1)
    idx = jnp.min(jnp.where(d == dmin, ids, big), axis=1, keepdims=True)
    onehot = (ids == idx).astype(jnp.float32)
    q_ref[...] = jnp.dot(onehot, emb, preferred_element_type=jnp.float32)


def _quantize(z_flat, emb):
    r, dim = z_flat.shape
    k = emb.shape[0]
    blk = 224
    z_sq = (z_flat ** 2).sum(1, keepdims=True)
    emb_sq = (emb ** 2).sum(1)[None, :]
    return pl.pallas_call(
        _vq_body,
        grid=(r // blk,),
        in_specs=[
            pl.BlockSpec((blk, dim), lambda i: (i, 0)),
            pl.BlockSpec((blk, 1), lambda i: (i, 0)),
            pl.BlockSpec((k, dim), lambda i: (0, 0)),
            pl.BlockSpec((1, k), lambda i: (0, 0)),
        ],
        out_specs=pl.BlockSpec((blk, dim), lambda i: (i, 0)),
        out_shape=jax.ShapeDtypeStruct((r, dim), jnp.float32),
    )(z_flat, z_sq, emb, emb_sq)


_KTAPS = ((1, 3), (2, 0))
_DELTA = {0: 1, 1: 0, 2: 0, 3: -1}


def _shift_rows(p, s, cout):
    if s > 0:
        return jnp.concatenate([p[s:], jnp.zeros((s, cout), jnp.float32)], 0)
    if s < 0:
        return jnp.concatenate([jnp.zeros((-s, cout), jnp.float32), p[:s]], 0)
    return p


def _dct_body(a_ref, w_ref, b_ref, ml_ref, mr_ref, o_ref, *, wdim, n, cout,
              act):
    a = a_ref[0]
    bias = b_ref[...]
    ml = ml_ref[...]
    mr = mr_ref[...]
    for r in range(2):
        for c in range(2):
            acc = bias * jnp.ones((n, 1), jnp.float32)
            for ky in _KTAPS[r]:
                for kx in _KTAPS[c]:
                    wk = w_ref[ky * 4 + kx]
                    p = jnp.dot(a, wk, preferred_element_type=jnp.float32)
                    ps = _shift_rows(p, _DELTA[ky] * wdim + _DELTA[kx], cout)
                    if _DELTA[kx] == 1:
                        ps = ps * mr
                    elif _DELTA[kx] == -1:
                        ps = ps * ml
                    acc = acc + ps
            o_ref[0, 2 * r + c] = act(acc)


def _masks(h, w):
    xs = jnp.arange(h * w, dtype=jnp.int32) % w
    ml = (xs != 0).astype(jnp.float32)[:, None]
    mr = (xs != w - 1).astype(jnp.float32)[:, None]
    return ml, mr


def _dconvt(a, dw, db, h, w, act):
    b, n, cin = a.shape
    cout = dw.shape[1]
    wm = jnp.transpose(dw, (2, 3, 0, 1)).reshape(16, cin, cout)
    bias = db[None, :]
    ml, mr = _masks(h, w)
    body = functools.partial(_dct_body, wdim=w, n=n, cout=cout, act=act)
    return pl.pallas_call(
        body,
        grid=(b,),
        in_specs=[
            pl.BlockSpec((1, n, cin), lambda i: (i, 0, 0)),
            pl.BlockSpec((16, cin, cout), lambda i: (0, 0, 0)),
            pl.BlockSpec((1, cout), lambda i: (0, 0)),
            pl.BlockSpec((n, 1), lambda i: (0, 0)),
            pl.BlockSpec((n, 1), lambda i: (0, 0)),
        ],
        out_specs=pl.BlockSpec((1, 4, n, cout), lambda i: (i, 0, 0, 0)),
        out_shape=jax.ShapeDtypeStruct((b, 4, n, cout), jnp.float32),
    )(a, wm, bias, ml, mr)


def _dct4_body(a_ref, w_ref, b_ref, ml_ref, mr_ref, o_ref, *, wdim, n, cout):
    a = a_ref[0]
    bias = b_ref[...]
    ml = ml_ref[...]
    mr = mr_ref[...]
    for r in range(2):
        for c in range(2):
            acc = bias * jnp.ones((1, n), jnp.float32)
            for ky in _KTAPS[r]:
                for kx in _KTAPS[c]:
                    wk = w_ref[ky * 4 + kx]
                    p = jnp.dot(wk, a, preferred_element_type=jnp.float32)
                    s = _DELTA[ky] * wdim + _DELTA[kx]
                    if s > 0:
                        ps = jnp.concatenate(
                            [p[:, s:], jnp.zeros((cout, s), jnp.float32)], 1)
                    elif s < 0:
                        ps = jnp.concatenate(
                            [jnp.zeros((cout, -s), jnp.float32), p[:, :s]], 1)
                    else:
                        ps = p
                    if _DELTA[kx] == 1:
                        ps = ps * mr
                    elif _DELTA[kx] == -1:
                        ps = ps * ml
                    acc = acc + ps
            o_ref[0, 2 * r + c] = jnp.tanh(acc)


def _dconvt4(a_t, dw, db, h, w):
    b, cin, n = a_t.shape
    cout = dw.shape[1]
    wm = jnp.transpose(dw, (2, 3, 1, 0)).reshape(16, cout, cin)
    bias = db[:, None]
    xs = jnp.arange(n, dtype=jnp.int32) % w
    ml = (xs != 0).astype(jnp.float32)[None, :]
    mr = (xs != w - 1).astype(jnp.float32)[None, :]
    body = functools.partial(_dct4_body, wdim=w, n=n, cout=cout)
    return pl.pallas_call(
        body,
        grid=(b,),
        in_specs=[
            pl.BlockSpec((1, cin, n), lambda i: (i, 0, 0)),
            pl.BlockSpec((16, cout, cin), lambda i: (0, 0, 0)),
            pl.BlockSpec((cout, 1), lambda i: (0, 0)),
            pl.BlockSpec((1, n), lambda i: (0, 0)),
            pl.BlockSpec((1, n), lambda i: (0, 0)),
        ],
        out_specs=pl.BlockSpec((1, 4, cout, n), lambda i: (i, 0, 0, 0)),
        out_shape=jax.ShapeDtypeStruct((b, 4, cout, n), jnp.float32),
    )(a_t, wm, bias, ml, mr)


def _interleave(o, h, w):
    b = o.shape[0]
    c = o.shape[3]
    o = o.reshape(b, 2, 2, h, w, c).transpose(0, 3, 1, 4, 2, 5)
    return o.reshape(b, 4 * h * w, c)


def kernel(x, w1, b1, w2, b2, w3, b3, w4, b4, emb,
           dw1, db1, dw2, db2, dw3, db3, dw4, db4):
    relu = jax.nn.relu
    z = relu(_conv(x, w1, b1, 2))
    z = relu(_conv(z, w2, b2, 2))
    z = relu(_conv(z, w3, b3, 2))
    z = relu(_conv(z, w4, b4, 2))
    zp = jnp.transpose(z, (0, 2, 3, 1))
    z_flat = zp.reshape(-1, emb.shape[1])
    q = _quantize(z_flat, emb)

    a = q.reshape(8, 196, 512)
    a = _interleave(_dconvt(a, dw1, db1, 14, 14, relu), 14, 14)
    a = _interleave(_dconvt(a, dw2, db2, 28, 28, relu), 28, 28)
    a = _interleave(_dconvt(a, dw3, db3, 56, 56, relu), 56, 56)
    a_t = jnp.transpose(a, (0, 2, 1))
    o4 = _dconvt4(a_t, dw4, db4, 112, 112)
    y = o4.reshape(8, 2, 2, 3, 112, 112).transpose(0, 3, 4, 1, 5, 2)
    return y.reshape(8, 3, 224, 224)


# speedup vs baseline: 1.9404x; 1.3184x over previous
import functools

import jax
import jax.numpy as jnp
from jax import lax
from jax.experimental import pallas as pl

_DN = ('NCHW', 'OIHW', 'NCHW')


def _conv(x, w, b, stride):
    y = lax.conv_general_dilated(x, w, (stride, stride), [(1, 1), (1, 1)],
                                 dimension_numbers=_DN)
    return y + b[None, :, None, None]


def _vq_body(zf_ref, z_sq_ref, emb_ref, emb_sq_ref, q_ref):
    zf = zf_ref[...]
    emb = emb_ref[...]
    z_sq = z_sq_ref[...]
    emb_sq = emb_sq_ref[...]
    zdote = lax.dot_general(zf, emb, (((1,), (1,)), ((), ())),
                            preferred_element_type=jnp.float32)
    d = (z_sq - 2.0 * zdote) + emb_sq
    dmin = jnp.min(d, axis=1, keepdims=True)
    ids = lax.broadcasted_iota(jnp.int32, d.shape, 1)
    big = jnp.int32(d.shape[1] + 1)
    idx = jnp.min(jnp.where(d == dmin, ids, big), axis=1, keepdims=True)
    onehot = (ids == idx).astype(jnp.float32)
    q_ref[...] = jnp.dot(onehot, emb, preferred_element_type=jnp.float32)


def _quantize(z_flat, emb):
    r, dim = z_flat.shape
    k = emb.shape[0]
    blk = 224
    z_sq = (z_flat ** 2).sum(1, keepdims=True)
    emb_sq = (emb ** 2).sum(1)[None, :]
    return pl.pallas_call(
        _vq_body,
        grid=(r // blk,),
        in_specs=[
            pl.BlockSpec((blk, dim), lambda i: (i, 0)),
            pl.BlockSpec((blk, 1), lambda i: (i, 0)),
            pl.BlockSpec((k, dim), lambda i: (0, 0)),
            pl.BlockSpec((1, k), lambda i: (0, 0)),
        ],
        out_specs=pl.BlockSpec((blk, dim), lambda i: (i, 0)),
        out_shape=jax.ShapeDtypeStruct((r, dim), jnp.float32),
    )(z_flat, z_sq, emb, emb_sq)


_KTAPS = ((1, 3), (2, 0))
_DELTA = {0: 1, 1: 0, 2: 0, 3: -1}
_W = 14
_N = 196


def _phase_body(a_ref, w_ref, b_ref, ml_ref, mr_ref, o_ref, *, lvl, cout,
                cpad, act):
    nplanes = 4 ** lvl
    half = 2 ** lvl
    bias = b_ref[...]
    ml = ml_ref[...]
    mr = mr_ref[...]
    wall = w_ref[...]
    pall = []
    for p in range(nplanes):
        a = a_ref[0, p]
        pall.append(jnp.dot(wall, a, preferred_element_type=jnp.float32))
    for yb_out in range(2 * half):
        r, yb = yb_out & 1, yb_out >> 1
        for xb_out in range(2 * half):
            c, xb = xb_out & 1, xb_out >> 1
            acc = None
            for ky in _KTAPS[r]:
                sy, ybm = divmod(yb + _DELTA[ky], half)
                for kx in _KTAPS[c]:
                    sx, xbm = divmod(xb + _DELTA[kx], half)
                    t = ky * 4 + kx
                    chunk = pall[ybm * half + xbm][t * cpad:t * cpad + cout]
                    s = sy * _W + sx
                    if s > 0:
                        chunk = jnp.concatenate(
                            [chunk[:, s:], jnp.zeros((cout, s), jnp.float32)],
                            1)
                    elif s < 0:
                        chunk = jnp.concatenate(
                            [jnp.zeros((cout, -s), jnp.float32), chunk[:, :s]],
                            1)
                    if sx == 1:
                        chunk = chunk * mr
                    elif sx == -1:
                        chunk = chunk * ml
                    acc = chunk if acc is None else acc + chunk
            o_ref[0, yb_out * 2 * half + xb_out] = act(acc + bias)


def _phase_layer(a, dw, db, lvl, act):
    b, nplanes, cin, n = a.shape
    cout = dw.shape[1]
    cpad = max(cout, 8)
    wt = jnp.transpose(dw, (2, 3, 1, 0)).reshape(16, cout, cin)
    if cpad != cout:
        wt = jnp.pad(wt, ((0, 0), (0, cpad - cout), (0, 0)))
    wall = wt.reshape(16 * cpad, cin)
    bias = db[:, None]
    xs = jnp.arange(n, dtype=jnp.int32) % _W
    ml = (xs != 0).astype(jnp.float32)[None, :]
    mr = (xs != _W - 1).astype(jnp.float32)[None, :]
    body = functools.partial(_phase_body, lvl=lvl, cout=cout, cpad=cpad,
                             act=act)
    return pl.pallas_call(
        body,
        grid=(b,),
        in_specs=[
            pl.BlockSpec((1, nplanes, cin, n), lambda i: (i, 0, 0, 0)),
            pl.BlockSpec((16 * cpad, cin), lambda i: (0, 0)),
            pl.BlockSpec((cout, 1), lambda i: (0, 0)),
            pl.BlockSpec((1, n), lambda i: (0, 0)),
            pl.BlockSpec((1, n), lambda i: (0, 0)),
        ],
        out_specs=pl.BlockSpec((1, 4 * nplanes, cout, n),
                               lambda i: (i, 0, 0, 0)),
        out_shape=jax.ShapeDtypeStruct((b, 4 * nplanes, cout, n),
                                       jnp.float32),
    )(a, wall, bias, ml, mr)


def kernel(x, w1, b1, w2, b2, w3, b3, w4, b4, emb,
           dw1, db1, dw2, db2, dw3, db3, dw4, db4):
    relu = jax.nn.relu
    z = relu(_conv(x, w1, b1, 2))
    z = relu(_conv(z, w2, b2, 2))
    z = relu(_conv(z, w3, b3, 2))
    z = relu(_conv(z, w4, b4, 2))
    zp = jnp.transpose(z, (0, 2, 3, 1))
    z_flat = zp.reshape(-1, emb.shape[1])
    q = _quantize(z_flat, emb)

    a = jnp.transpose(q.reshape(8, _N, 512), (0, 2, 1)).reshape(8, 1, 512, _N)
    a = _phase_layer(a, dw1, db1, 0, relu)
    a = _phase_layer(a, dw2, db2, 1, relu)
    a = _phase_layer(a, dw3, db3, 2, relu)
    a = _phase_layer(a, dw4, db4, 3, jnp.tanh)
    y = a.reshape(8, 16, 16, 3, _W, _W).transpose(0, 3, 4, 1, 5, 2)
    return y.reshape(8, 3, 224, 224)
